# baseline (speedup 1.0000x reference)
import functools

import jax
import jax.numpy as jnp
from jax import lax
from jax.experimental import pallas as pl
from jax.experimental.pallas import tpu as pltpu
from jax.experimental.pallas import tpu_sc as plsc

N = 2048
D = 768
F = 3072
E = 8

TB = 256
NB = N // TB
BK = 256
NBLK = N * 2 // BK + E
M = NBLK * BK

NW = 32
CH = N // NW
L = 16


def _router_kernel(x_ref, gw_ref, gb_ref,
                   d0_ref, d1_ref, g0_ref, g1_ref, sca_ref,
                   carry_ref, i0s_ref, i1s_ref, r0s_ref, r1s_ref):
    j = pl.program_id(0)
    xb = x_ref[...]
    logits = jnp.dot(xb, gw_ref[...],
                     preferred_element_type=jnp.float32) + gb_ref[0, :]
    cols = jax.lax.broadcasted_iota(jnp.int32, (TB, E), 1)
    m1 = jnp.max(logits, axis=1, keepdims=True)
    i1 = jnp.min(jnp.where(logits == m1, cols, E), axis=1, keepdims=True)
    masked = jnp.where(cols == i1, -jnp.inf, logits)
    m2 = jnp.max(masked, axis=1, keepdims=True)
    i2 = jnp.min(jnp.where(masked == m2, cols, E), axis=1, keepdims=True)
    s = jnp.exp(m2 - m1)
    gg0 = 1.0 / (1.0 + s)
    gg1 = s / (1.0 + s)
    mask = ((cols == i1) | (cols == i2)).astype(jnp.float32)

    rit = jax.lax.broadcasted_iota(jnp.int32, (TB, TB), 0)
    cit = jax.lax.broadcasted_iota(jnp.int32, (TB, TB), 1)
    tri = (cit < rit).astype(jnp.float32)
    ranks = jnp.dot(tri, mask, preferred_element_type=jnp.float32)

    @pl.when(j == 0)
    def _():
        carry_ref[...] = jnp.zeros_like(carry_ref)

    rg = ranks + carry_ref[0, :]
    r0 = jnp.sum(rg * (cols == i1), axis=1, keepdims=True)
    r1 = jnp.sum(rg * (cols == i2), axis=1, keepdims=True)
    carry_ref[...] = carry_ref[...] + jnp.sum(mask, axis=0, keepdims=True)

    g0_ref[...] = jnp.broadcast_to(gg0, (TB, L))
    g1_ref[...] = jnp.broadcast_to(gg1, (TB, L))
    sl = pl.ds(j * TB, TB)
    i0s_ref[sl, :] = i1
    i1s_ref[sl, :] = i2
    r0s_ref[sl, :] = r0.astype(jnp.int32)
    r1s_ref[sl, :] = r1.astype(jnp.int32)

    @pl.when(j == NB - 1)
    def _():
        counts = carry_ref[...].astype(jnp.int32)
        nblk = (counts + (BK - 1)) // BK
        eit = jax.lax.broadcasted_iota(jnp.int32, (E, E), 0)
        ejt = jax.lax.broadcasted_iota(jnp.int32, (E, E), 1)
        tri8 = (eit < ejt).astype(jnp.float32)
        blk_start = jnp.dot(nblk.astype(jnp.float32), tri8,
                            preferred_element_type=jnp.float32
                            ).astype(jnp.int32)
        poff = blk_start * BK
        sca_ref[0, pl.ds(0, E)] = poff[0, :]
        sca_ref[0, pl.ds(E, E)] = nblk[0, :]

        def cbody(c, _):
            csl = pl.ds(c * TB, TB)
            p0 = jnp.sum(jnp.where(i0s_ref[csl, :] == cols, poff[0, :], 0),
                         axis=1, keepdims=True)
            p1 = jnp.sum(jnp.where(i1s_ref[csl, :] == cols, poff[0, :], 0),
                         axis=1, keepdims=True)
            d0_ref[csl, :] = r0s_ref[csl, :] + p0
            d1_ref[csl, :] = r1s_ref[csl, :] + p1
            return 0

        jax.lax.fori_loop(0, NB, cbody, 0)


def _router(x, gate_w, gb):
    vspec = pl.BlockSpec((TB, 1), lambda j: (j, 0))
    full = pl.BlockSpec((N, 1), lambda j: (0, 0))
    return pl.pallas_call(
        _router_kernel,
        grid=(NB,),
        in_specs=[
            pl.BlockSpec((TB, D), lambda j: (j, 0)),
            pl.BlockSpec((D, E), lambda j: (0, 0)),
            pl.BlockSpec((1, E), lambda j: (0, 0)),
        ],
        out_specs=[full, full,
                   pl.BlockSpec((TB, L), lambda j: (j, 0)),
                   pl.BlockSpec((TB, L), lambda j: (j, 0)),
                   pl.BlockSpec((1, 2 * E), lambda j: (0, 0))],
        out_shape=[
            jax.ShapeDtypeStruct((N, 1), jnp.int32),
            jax.ShapeDtypeStruct((N, 1), jnp.int32),
            jax.ShapeDtypeStruct((N, L), jnp.float32),
            jax.ShapeDtypeStruct((N, L), jnp.float32),
            jax.ShapeDtypeStruct((1, 2 * E), jnp.int32),
        ],
        scratch_shapes=[pltpu.VMEM((1, E), jnp.float32),
                        pltpu.VMEM((N, 1), jnp.int32),
                        pltpu.VMEM((N, 1), jnp.int32),
                        pltpu.VMEM((N, 1), jnp.int32),
                        pltpu.VMEM((N, 1), jnp.int32)],
    )(x, gate_w, gb)


_SC_MESH = plsc.VectorSubcoreMesh(core_axis_name="c", subcore_axis_name="s")


@functools.partial(
    pl.kernel, mesh=_SC_MESH,
    out_type=jax.ShapeDtypeStruct((M, D), jnp.float32),
    scratch_types=[
        pltpu.VMEM((CH, D), jnp.float32),
        pltpu.VMEM((CH,), jnp.int32),
        pltpu.VMEM((CH,), jnp.int32),
        pltpu.SemaphoreType.DMA,
    ])
def _dispatch(x_hbm, d0_hbm, d1_hbm, xg_hbm, xv, d0v, d1v, sem):
    wid = lax.axis_index("s") * 2 + lax.axis_index("c")
    base = wid * CH
    pltpu.sync_copy(x_hbm.at[pl.ds(base, CH)], xv)
    pltpu.sync_copy(d0_hbm.at[pl.ds(base, CH)], d0v)
    pltpu.sync_copy(d1_hbm.at[pl.ds(base, CH)], d1v)
    c0 = pltpu.async_copy(xv, xg_hbm.at[d0v], sem)
    c1 = pltpu.async_copy(xv, xg_hbm.at[d1v], sem)
    c0.wait()
    c1.wait()


def _gelu_exact(v):
    return 0.5 * v * (1.0 + jax.lax.erf(v * 0.7071067811865476))


NS = 3
FS = F // NS
MAXCB = N // BK


def _xg_rows_copy(sref, xg_hbm, xgv_ref, sem, e):
    poff_e = pl.multiple_of(sref[0, e], BK)
    nb_e = sref[0, E + e]
    copies = []
    for cb in range(MAXCB):
        rows = pl.ds(poff_e + cb * BK, BK)
        copies.append((cb < nb_e,
                       pltpu.make_async_copy(xg_hbm.at[rows, :],
                                             xgv_ref.at[rows, :], sem)))
    return copies


def _og_rows_copy(sref, ogv_ref, og_hbm, sem, e):
    poff_e = pl.multiple_of(sref[0, e], BK)
    nb_e = sref[0, E + e]
    copies = []
    for cb in range(MAXCB):
        rows = pl.ds(poff_e + cb * BK, BK)
        copies.append((cb < nb_e,
                       pltpu.make_async_copy(ogv_ref.at[rows, :],
                                             og_hbm.at[rows, :], sem)))
    return copies


def _start_guarded(copies):
    for cond, cp in copies:
        @pl.when(cond)
        def _():
            cp.start()


def _wait_guarded(copies):
    for cond, cp in copies:
        @pl.when(cond)
        def _():
            cp.wait()


def _group_mlp_kernel(sref, xg_hbm, w1_ref, b1_ref, w2_ref, b2_ref, og_hbm,
                      xgv_ref, ogv_ref, sem_in, sem_out):
    e = pl.program_id(0)
    s = pl.program_id(1)
    poff_e = pl.multiple_of(sref[0, e], BK)
    nb_e = sref[0, E + e]

    @pl.when(jnp.logical_and(e == 0, s == 0))
    def _():
        _start_guarded(_xg_rows_copy(sref, xg_hbm, xgv_ref, sem_in, 0))

    @pl.when(s == 0)
    def _():
        _wait_guarded(_xg_rows_copy(sref, xg_hbm, xgv_ref, sem_in, e))

        @pl.when(e < E - 1)
        def _():
            _start_guarded(_xg_rows_copy(sref, xg_hbm, xgv_ref, sem_in, e + 1))

    for cb in range(MAXCB):
        @pl.when(cb < nb_e)
        def _():
            rows = pl.ds(poff_e + cb * BK, BK)
            hs = jnp.dot(xgv_ref[rows, :], w1_ref[0],
                         preferred_element_type=jnp.float32) + b1_ref[0, 0]
            hs = _gelu_exact(hs)
            contrib = jnp.dot(hs, w2_ref[0],
                              preferred_element_type=jnp.float32)

            @pl.when(s == 0)
            def _():
                ogv_ref[rows, :] = contrib + b2_ref[0, 0]

            @pl.when(s > 0)
            def _():
                ogv_ref[rows, :] = ogv_ref[rows, :] + contrib

    @pl.when(s == NS - 1)
    def _():
        @pl.when(e > 0)
        def _():
            _wait_guarded(_og_rows_copy(sref, ogv_ref, og_hbm, sem_out, e - 1))

        _start_guarded(_og_rows_copy(sref, ogv_ref, og_hbm, sem_out, e))

        @pl.when(e == E - 1)
        def _():
            _wait_guarded(_og_rows_copy(sref, ogv_ref, og_hbm, sem_out, e))


def _group_mlp(scalars, xg, w1, b1r, w2, b2r):
    return pl.pallas_call(
        _group_mlp_kernel,
        grid_spec=pltpu.PrefetchScalarGridSpec(
            num_scalar_prefetch=1,
            grid=(E, NS),
            in_specs=[
                pl.BlockSpec(memory_space=pl.ANY),
                pl.BlockSpec((1, D, FS), lambda e, s, sc: (e, 0, s)),
                pl.BlockSpec((1, 1, FS), lambda e, s, sc: (e, 0, s)),
                pl.BlockSpec((1, FS, D), lambda e, s, sc: (e, s, 0)),
                pl.BlockSpec((1, 1, D), lambda e, s, sc: (e, 0, 0)),
            ],
            out_specs=pl.BlockSpec(memory_space=pl.ANY),
            scratch_shapes=[pltpu.VMEM((M, D), jnp.float32),
                            pltpu.VMEM((M, D), jnp.float32),
                            pltpu.SemaphoreType.DMA,
                            pltpu.SemaphoreType.DMA],
        ),
        out_shape=jax.ShapeDtypeStruct((M, D), jnp.float32),
    )(scalars, xg, w1, b1r, w2, b2r)


@functools.partial(
    pl.kernel, mesh=_SC_MESH,
    out_type=jax.ShapeDtypeStruct((N, D), jnp.float32),
    scratch_types=[
        pltpu.VMEM((CH, D), jnp.float32),
        pltpu.VMEM((CH, D), jnp.float32),
        pltpu.VMEM((CH,), jnp.int32),
        pltpu.VMEM((CH,), jnp.int32),
        pltpu.VMEM((CH, L), jnp.float32),
        pltpu.VMEM((CH, L), jnp.float32),
        pltpu.SemaphoreType.DMA,
    ])
def _gather_combine(og_hbm, d0_hbm, d1_hbm, g0_hbm, g1_hbm, out_hbm,
                    r0v, r1v, d0v, d1v, g0v, g1v, sem):
    wid = lax.axis_index("s") * 2 + lax.axis_index("c")
    base = wid * CH
    pltpu.sync_copy(d0_hbm.at[pl.ds(base, CH)], d0v)
    pltpu.sync_copy(d1_hbm.at[pl.ds(base, CH)], d1v)
    pltpu.sync_copy(g0_hbm.at[pl.ds(base, CH)], g0v)
    pltpu.sync_copy(g1_hbm.at[pl.ds(base, CH)], g1v)
    c0 = pltpu.async_copy(og_hbm.at[d0v], r0v, sem)
    c1 = pltpu.async_copy(og_hbm.at[d1v], r1v, sem)
    c0.wait()
    c1.wait()

    def row_body(r, _):
        ga = g0v[r, :]
        gb = g1v[r, :]
        for c in range(D // L):
            sl = pl.ds(c * L, L)
            r0v[r, sl] = ga * r0v[r, sl] + gb * r1v[r, sl]
        return 0

    jax.lax.fori_loop(0, CH, row_body, 0)
    pltpu.sync_copy(r0v, out_hbm.at[pl.ds(base, CH)])


@jax.jit
def kernel(x, gate_w, gate_b, w1, b1, w2, b2):
    gb = gate_b.reshape(1, E)
    b1r = b1.reshape(E, 1, F)
    b2r = b2.reshape(E, 1, D)

    d0, d1, g0, g1, scalars = _router(x, gate_w, gb)
    d0f, d1f = d0.reshape(N), d1.reshape(N)

    xg = _dispatch(x, d0f, d1f)
    og = _group_mlp(scalars, xg, w1, b1r, w2, b2r)
    out = _gather_combine(og, d0f, d1f, g0, g1)
    return out


# speedup vs baseline: 1.0764x; 1.0764x over previous
import functools

import jax
import jax.numpy as jnp
from jax import lax
from jax.experimental import pallas as pl
from jax.experimental.pallas import tpu as pltpu
from jax.experimental.pallas import tpu_sc as plsc

N = 2048
D = 768
F = 3072
E = 8

TB = 256
NB = N // TB
BK = 256
NBLK = N * 2 // BK + E
M = NBLK * BK

NW = 32
CH = N // NW
L = 16


def _router_kernel(x_ref, gw_ref, gb_ref,
                   d0_ref, d1_ref, g0_ref, g1_ref, sca_ref,
                   carry_ref, i0s_ref, i1s_ref, r0s_ref, r1s_ref):
    j = pl.program_id(0)
    xb = x_ref[...]
    logits = jnp.dot(xb, gw_ref[...],
                     preferred_element_type=jnp.float32) + gb_ref[0, :]
    cols = jax.lax.broadcasted_iota(jnp.int32, (TB, E), 1)
    m1 = jnp.max(logits, axis=1, keepdims=True)
    i1 = jnp.min(jnp.where(logits == m1, cols, E), axis=1, keepdims=True)
    masked = jnp.where(cols == i1, -jnp.inf, logits)
    m2 = jnp.max(masked, axis=1, keepdims=True)
    i2 = jnp.min(jnp.where(masked == m2, cols, E), axis=1, keepdims=True)
    s = jnp.exp(m2 - m1)
    gg0 = 1.0 / (1.0 + s)
    gg1 = s / (1.0 + s)
    mask = ((cols == i1) | (cols == i2)).astype(jnp.float32)

    rit = jax.lax.broadcasted_iota(jnp.int32, (TB, TB), 0)
    cit = jax.lax.broadcasted_iota(jnp.int32, (TB, TB), 1)
    tri = (cit < rit).astype(jnp.float32)
    ranks = jnp.dot(tri, mask, preferred_element_type=jnp.float32)

    @pl.when(j == 0)
    def _():
        carry_ref[...] = jnp.zeros_like(carry_ref)

    rg = ranks + carry_ref[0, :]
    r0 = jnp.sum(rg * (cols == i1), axis=1, keepdims=True)
    r1 = jnp.sum(rg * (cols == i2), axis=1, keepdims=True)
    carry_ref[...] = carry_ref[...] + jnp.sum(mask, axis=0, keepdims=True)

    g0_ref[...] = jnp.broadcast_to(gg0, (TB, L))
    g1_ref[...] = jnp.broadcast_to(gg1, (TB, L))
    sl = pl.ds(j * TB, TB)
    i0s_ref[sl, :] = i1
    i1s_ref[sl, :] = i2
    r0s_ref[sl, :] = r0.astype(jnp.int32)
    r1s_ref[sl, :] = r1.astype(jnp.int32)

    @pl.when(j == NB - 1)
    def _():
        counts = carry_ref[...].astype(jnp.int32)
        nblk = (counts + (BK - 1)) // BK
        eit = jax.lax.broadcasted_iota(jnp.int32, (E, E), 0)
        ejt = jax.lax.broadcasted_iota(jnp.int32, (E, E), 1)
        tri8 = (eit < ejt).astype(jnp.float32)
        blk_start = jnp.dot(nblk.astype(jnp.float32), tri8,
                            preferred_element_type=jnp.float32
                            ).astype(jnp.int32)
        poff = blk_start * BK
        sca_ref[0, pl.ds(0, E)] = poff[0, :]
        sca_ref[0, pl.ds(E, E)] = nblk[0, :]

        def cbody(c, _):
            csl = pl.ds(c * TB, TB)
            p0 = jnp.sum(jnp.where(i0s_ref[csl, :] == cols, poff[0, :], 0),
                         axis=1, keepdims=True)
            p1 = jnp.sum(jnp.where(i1s_ref[csl, :] == cols, poff[0, :], 0),
                         axis=1, keepdims=True)
            d0_ref[csl, :] = r0s_ref[csl, :] + p0
            d1_ref[csl, :] = r1s_ref[csl, :] + p1
            return 0

        jax.lax.fori_loop(0, NB, cbody, 0)


def _router(x, gate_w, gb):
    vspec = pl.BlockSpec((TB, 1), lambda j: (j, 0))
    full = pl.BlockSpec((N, 1), lambda j: (0, 0))
    return pl.pallas_call(
        _router_kernel,
        grid=(NB,),
        in_specs=[
            pl.BlockSpec((TB, D), lambda j: (j, 0)),
            pl.BlockSpec((D, E), lambda j: (0, 0)),
            pl.BlockSpec((1, E), lambda j: (0, 0)),
        ],
        out_specs=[full, full,
                   pl.BlockSpec((TB, L), lambda j: (j, 0)),
                   pl.BlockSpec((TB, L), lambda j: (j, 0)),
                   pl.BlockSpec((1, 2 * E), lambda j: (0, 0))],
        out_shape=[
            jax.ShapeDtypeStruct((N, 1), jnp.int32),
            jax.ShapeDtypeStruct((N, 1), jnp.int32),
            jax.ShapeDtypeStruct((N, L), jnp.float32),
            jax.ShapeDtypeStruct((N, L), jnp.float32),
            jax.ShapeDtypeStruct((1, 2 * E), jnp.int32),
        ],
        scratch_shapes=[pltpu.VMEM((1, E), jnp.float32),
                        pltpu.VMEM((N, 1), jnp.int32),
                        pltpu.VMEM((N, 1), jnp.int32),
                        pltpu.VMEM((N, 1), jnp.int32),
                        pltpu.VMEM((N, 1), jnp.int32)],
    )(x, gate_w, gb)


_SC_MESH = plsc.VectorSubcoreMesh(core_axis_name="c", subcore_axis_name="s")


@functools.partial(
    pl.kernel, mesh=_SC_MESH,
    out_type=jax.ShapeDtypeStruct((M, D), jnp.float32),
    scratch_types=[
        pltpu.VMEM((CH, D), jnp.float32),
        pltpu.VMEM((CH,), jnp.int32),
        pltpu.VMEM((CH,), jnp.int32),
        pltpu.SemaphoreType.DMA,
    ])
def _dispatch(x_hbm, d0_hbm, d1_hbm, xg_hbm, xv, d0v, d1v, sem):
    wid = lax.axis_index("s") * 2 + lax.axis_index("c")
    base = wid * CH
    pltpu.sync_copy(x_hbm.at[pl.ds(base, CH)], xv)
    pltpu.sync_copy(d0_hbm.at[pl.ds(base, CH)], d0v)
    pltpu.sync_copy(d1_hbm.at[pl.ds(base, CH)], d1v)
    c0 = pltpu.async_copy(xv, xg_hbm.at[d0v], sem)
    c1 = pltpu.async_copy(xv, xg_hbm.at[d1v], sem)
    c0.wait()
    c1.wait()


def _gelu_exact(v):
    return 0.5 * v * (1.0 + jax.lax.erf(v * 0.7071067811865476))


NS = 2
FS = F // NS
MAXCB = N // BK


def _xg_rows_copy(sref, xg_hbm, xgv_ref, sem, e):
    poff_e = pl.multiple_of(sref[0, e], BK)
    nb_e = sref[0, E + e]
    copies = []
    for cb in range(MAXCB):
        rows = pl.ds(poff_e + cb * BK, BK)
        copies.append((cb < nb_e,
                       pltpu.make_async_copy(xg_hbm.at[rows, :],
                                             xgv_ref.at[rows, :], sem)))
    return copies


def _og_rows_copy(sref, ogv_ref, og_hbm, sem, e):
    poff_e = pl.multiple_of(sref[0, e], BK)
    nb_e = sref[0, E + e]
    copies = []
    for cb in range(MAXCB):
        rows = pl.ds(poff_e + cb * BK, BK)
        copies.append((cb < nb_e,
                       pltpu.make_async_copy(ogv_ref.at[rows, :],
                                             og_hbm.at[rows, :], sem)))
    return copies


def _start_guarded(copies):
    for cond, cp in copies:
        @pl.when(cond)
        def _():
            cp.start()


def _wait_guarded(copies):
    for cond, cp in copies:
        @pl.when(cond)
        def _():
            cp.wait()


def _group_mlp_kernel(sref, xg_hbm, w1_ref, b1_ref, w2_ref, b2_ref, og_hbm,
                      xgv_ref, ogv_ref, sem_in, sem_out):
    e = pl.program_id(0)
    s = pl.program_id(1)
    poff_e = pl.multiple_of(sref[0, e], BK)
    nb_e = sref[0, E + e]

    @pl.when(jnp.logical_and(e == 0, s == 0))
    def _():
        _start_guarded(_xg_rows_copy(sref, xg_hbm, xgv_ref, sem_in, 0))

    @pl.when(s == 0)
    def _():
        _wait_guarded(_xg_rows_copy(sref, xg_hbm, xgv_ref, sem_in, e))

        @pl.when(e < E - 1)
        def _():
            _start_guarded(_xg_rows_copy(sref, xg_hbm, xgv_ref, sem_in, e + 1))

    for cb in range(MAXCB):
        @pl.when(cb < nb_e)
        def _():
            rows = pl.ds(poff_e + cb * BK, BK)
            hs = jnp.dot(xgv_ref[rows, :], w1_ref[0],
                         preferred_element_type=jnp.float32) + b1_ref[0, 0]
            hs = _gelu_exact(hs)
            contrib = jnp.dot(hs, w2_ref[0],
                              preferred_element_type=jnp.float32)

            @pl.when(s == 0)
            def _():
                ogv_ref[rows, :] = contrib + b2_ref[0, 0]

            @pl.when(s > 0)
            def _():
                ogv_ref[rows, :] = ogv_ref[rows, :] + contrib

    @pl.when(s == NS - 1)
    def _():
        @pl.when(e > 0)
        def _():
            _wait_guarded(_og_rows_copy(sref, ogv_ref, og_hbm, sem_out, e - 1))

        _start_guarded(_og_rows_copy(sref, ogv_ref, og_hbm, sem_out, e))

        @pl.when(e == E - 1)
        def _():
            _wait_guarded(_og_rows_copy(sref, ogv_ref, og_hbm, sem_out, e))


def _group_mlp(scalars, xg, w1, b1r, w2, b2r):
    return pl.pallas_call(
        _group_mlp_kernel,
        grid_spec=pltpu.PrefetchScalarGridSpec(
            num_scalar_prefetch=1,
            grid=(E, NS),
            in_specs=[
                pl.BlockSpec(memory_space=pl.ANY),
                pl.BlockSpec((1, D, FS), lambda e, s, sc: (e, 0, s)),
                pl.BlockSpec((1, 1, FS), lambda e, s, sc: (e, 0, s)),
                pl.BlockSpec((1, FS, D), lambda e, s, sc: (e, s, 0)),
                pl.BlockSpec((1, 1, D), lambda e, s, sc: (e, 0, 0)),
            ],
            out_specs=pl.BlockSpec(memory_space=pl.ANY),
            scratch_shapes=[pltpu.VMEM((M, D), jnp.float32),
                            pltpu.VMEM((M, D), jnp.float32),
                            pltpu.SemaphoreType.DMA,
                            pltpu.SemaphoreType.DMA],
        ),
        out_shape=jax.ShapeDtypeStruct((M, D), jnp.float32),
    )(scalars, xg, w1, b1r, w2, b2r)


@functools.partial(
    pl.kernel, mesh=_SC_MESH,
    out_type=jax.ShapeDtypeStruct((N, D), jnp.float32),
    scratch_types=[
        pltpu.VMEM((CH, D), jnp.float32),
        pltpu.VMEM((CH, D), jnp.float32),
        pltpu.VMEM((CH,), jnp.int32),
        pltpu.VMEM((CH,), jnp.int32),
        pltpu.VMEM((CH, L), jnp.float32),
        pltpu.VMEM((CH, L), jnp.float32),
        pltpu.SemaphoreType.DMA,
    ])
def _gather_combine(og_hbm, d0_hbm, d1_hbm, g0_hbm, g1_hbm, out_hbm,
                    r0v, r1v, d0v, d1v, g0v, g1v, sem):
    wid = lax.axis_index("s") * 2 + lax.axis_index("c")
    base = wid * CH
    HH = CH // 2
    pltpu.sync_copy(d0_hbm.at[pl.ds(base, CH)], d0v)
    pltpu.sync_copy(d1_hbm.at[pl.ds(base, CH)], d1v)
    pltpu.sync_copy(g0_hbm.at[pl.ds(base, CH)], g0v)
    pltpu.sync_copy(g1_hbm.at[pl.ds(base, CH)], g1v)
    copies = []
    for hb in range(2):
        hsl = pl.ds(hb * HH, HH)
        copies.append((pltpu.async_copy(og_hbm.at[d0v.at[hsl]],
                                        r0v.at[hsl], sem),
                       pltpu.async_copy(og_hbm.at[d1v.at[hsl]],
                                        r1v.at[hsl], sem)))

    def row_body(r, _):
        ga = g0v[r, :]
        gb = g1v[r, :]
        for c in range(D // L):
            sl = pl.ds(c * L, L)
            r0v[r, sl] = ga * r0v[r, sl] + gb * r1v[r, sl]
        return 0

    outcp = []
    for hb in range(2):
        copies[hb][0].wait()
        copies[hb][1].wait()
        jax.lax.fori_loop(hb * HH, (hb + 1) * HH, row_body, 0)
        outcp.append(pltpu.async_copy(r0v.at[pl.ds(hb * HH, HH)],
                                      out_hbm.at[pl.ds(base + hb * HH, HH)],
                                      sem))
    outcp[0].wait()
    outcp[1].wait()


@jax.jit
def kernel(x, gate_w, gate_b, w1, b1, w2, b2):
    gb = gate_b.reshape(1, E)
    b1r = b1.reshape(E, 1, F)
    b2r = b2.reshape(E, 1, D)

    d0, d1, g0, g1, scalars = _router(x, gate_w, gb)
    d0f, d1f = d0.reshape(N), d1.reshape(N)

    xg = _dispatch(x, d0f, d1f)
    og = _group_mlp(scalars, xg, w1, b1r, w2, b2r)
    out = _gather_combine(og, d0f, d1f, g0, g1)
    return out


# speedup vs baseline: 1.0804x; 1.0036x over previous
import functools

import jax
import jax.numpy as jnp
from jax import lax
from jax.experimental import pallas as pl
from jax.experimental.pallas import tpu as pltpu
from jax.experimental.pallas import tpu_sc as plsc

N = 2048
D = 768
F = 3072
E = 8

TB = 256
NB = N // TB
BK = 256
NBLK = N * 2 // BK + E
M = NBLK * BK

NW = 32
CH = N // NW
L = 16


def _router_kernel(x_ref, gw_ref, gb_ref,
                   d0_ref, d1_ref, g0_ref, g1_ref, sca_ref,
                   carry_ref, i0s_ref, i1s_ref, r0s_ref, r1s_ref):
    j = pl.program_id(0)
    xb = x_ref[...]
    logits = jnp.dot(xb, gw_ref[...],
                     preferred_element_type=jnp.float32) + gb_ref[0, :]
    cols = jax.lax.broadcasted_iota(jnp.int32, (TB, E), 1)
    m1 = jnp.max(logits, axis=1, keepdims=True)
    i1 = jnp.min(jnp.where(logits == m1, cols, E), axis=1, keepdims=True)
    masked = jnp.where(cols == i1, -jnp.inf, logits)
    m2 = jnp.max(masked, axis=1, keepdims=True)
    i2 = jnp.min(jnp.where(masked == m2, cols, E), axis=1, keepdims=True)
    s = jnp.exp(m2 - m1)
    gg0 = 1.0 / (1.0 + s)
    gg1 = s / (1.0 + s)
    mask = ((cols == i1) | (cols == i2)).astype(jnp.float32)

    rit = jax.lax.broadcasted_iota(jnp.int32, (TB, TB), 0)
    cit = jax.lax.broadcasted_iota(jnp.int32, (TB, TB), 1)
    tri = (cit < rit).astype(jnp.float32)
    ranks = jnp.dot(tri, mask, preferred_element_type=jnp.float32)

    @pl.when(j == 0)
    def _():
        carry_ref[...] = jnp.zeros_like(carry_ref)

    rg = ranks + carry_ref[0, :]
    r0 = jnp.sum(rg * (cols == i1), axis=1, keepdims=True)
    r1 = jnp.sum(rg * (cols == i2), axis=1, keepdims=True)
    carry_ref[...] = carry_ref[...] + jnp.sum(mask, axis=0, keepdims=True)

    g0_ref[...] = jnp.broadcast_to(gg0, (TB, L))
    g1_ref[...] = jnp.broadcast_to(gg1, (TB, L))
    sl = pl.ds(j * TB, TB)
    i0s_ref[sl, :] = i1
    i1s_ref[sl, :] = i2
    r0s_ref[sl, :] = r0.astype(jnp.int32)
    r1s_ref[sl, :] = r1.astype(jnp.int32)

    @pl.when(j == NB - 1)
    def _():
        counts = carry_ref[...].astype(jnp.int32)
        nblk = (counts + (BK - 1)) // BK
        eit = jax.lax.broadcasted_iota(jnp.int32, (E, E), 0)
        ejt = jax.lax.broadcasted_iota(jnp.int32, (E, E), 1)
        tri8 = (eit < ejt).astype(jnp.float32)
        blk_start = jnp.dot(nblk.astype(jnp.float32), tri8,
                            preferred_element_type=jnp.float32
                            ).astype(jnp.int32)
        poff = blk_start * BK
        sca_ref[0, pl.ds(0, E)] = poff[0, :]
        sca_ref[0, pl.ds(E, E)] = nblk[0, :]

        def cbody(c, _):
            csl = pl.ds(c * TB, TB)
            p0 = jnp.sum(jnp.where(i0s_ref[csl, :] == cols, poff[0, :], 0),
                         axis=1, keepdims=True)
            p1 = jnp.sum(jnp.where(i1s_ref[csl, :] == cols, poff[0, :], 0),
                         axis=1, keepdims=True)
            d0_ref[csl, :] = r0s_ref[csl, :] + p0
            d1_ref[csl, :] = r1s_ref[csl, :] + p1
            return 0

        jax.lax.fori_loop(0, NB, cbody, 0)


def _router(x, gate_w, gb):
    vspec = pl.BlockSpec((TB, 1), lambda j: (j, 0))
    full = pl.BlockSpec((N, 1), lambda j: (0, 0))
    return pl.pallas_call(
        _router_kernel,
        grid=(NB,),
        in_specs=[
            pl.BlockSpec((TB, D), lambda j: (j, 0)),
            pl.BlockSpec((D, E), lambda j: (0, 0)),
            pl.BlockSpec((1, E), lambda j: (0, 0)),
        ],
        out_specs=[full, full,
                   pl.BlockSpec((TB, L), lambda j: (j, 0)),
                   pl.BlockSpec((TB, L), lambda j: (j, 0)),
                   pl.BlockSpec((1, 2 * E), lambda j: (0, 0))],
        out_shape=[
            jax.ShapeDtypeStruct((N, 1), jnp.int32),
            jax.ShapeDtypeStruct((N, 1), jnp.int32),
            jax.ShapeDtypeStruct((N, L), jnp.float32),
            jax.ShapeDtypeStruct((N, L), jnp.float32),
            jax.ShapeDtypeStruct((1, 2 * E), jnp.int32),
        ],
        scratch_shapes=[pltpu.VMEM((1, E), jnp.float32),
                        pltpu.VMEM((N, 1), jnp.int32),
                        pltpu.VMEM((N, 1), jnp.int32),
                        pltpu.VMEM((N, 1), jnp.int32),
                        pltpu.VMEM((N, 1), jnp.int32)],
    )(x, gate_w, gb)


_SC_MESH = plsc.VectorSubcoreMesh(core_axis_name="c", subcore_axis_name="s")


@functools.partial(
    pl.kernel, mesh=_SC_MESH,
    out_type=jax.ShapeDtypeStruct((M, D), jnp.float32),
    scratch_types=[
        pltpu.VMEM((CH, D), jnp.float32),
        pltpu.VMEM((CH,), jnp.int32),
        pltpu.VMEM((CH,), jnp.int32),
        pltpu.SemaphoreType.DMA,
    ])
def _dispatch(x_hbm, d0_hbm, d1_hbm, xg_hbm, xv, d0v, d1v, sem):
    wid = lax.axis_index("s") * 2 + lax.axis_index("c")
    base = wid * CH
    pltpu.sync_copy(x_hbm.at[pl.ds(base, CH)], xv)
    pltpu.sync_copy(d0_hbm.at[pl.ds(base, CH)], d0v)
    pltpu.sync_copy(d1_hbm.at[pl.ds(base, CH)], d1v)
    c0 = pltpu.async_copy(xv, xg_hbm.at[d0v], sem)
    c1 = pltpu.async_copy(xv, xg_hbm.at[d1v], sem)
    c0.wait()
    c1.wait()


def _gelu_exact(v):
    return 0.5 * v * (1.0 + jax.lax.erf(v * 0.7071067811865476))


NS = 2
FS = F // NS
MAXCB = N // BK


def _xg_rows_copy(sref, xg_hbm, xgv_ref, sem, e):
    poff_e = pl.multiple_of(sref[0, e], BK)
    nb_e = sref[0, E + e]
    copies = []
    for cb in range(MAXCB):
        rows = pl.ds(poff_e + cb * BK, BK)
        copies.append((cb < nb_e,
                       pltpu.make_async_copy(xg_hbm.at[rows, :],
                                             xgv_ref.at[rows, :], sem)))
    return copies


def _og_rows_copy(sref, ogv_ref, og_hbm, sem, e):
    poff_e = pl.multiple_of(sref[0, e], BK)
    nb_e = sref[0, E + e]
    copies = []
    for cb in range(MAXCB):
        rows = pl.ds(poff_e + cb * BK, BK)
        copies.append((cb < nb_e,
                       pltpu.make_async_copy(ogv_ref.at[rows, :],
                                             og_hbm.at[rows, :], sem)))
    return copies


def _start_guarded(copies):
    for cond, cp in copies:
        @pl.when(cond)
        def _():
            cp.start()


def _wait_guarded(copies):
    for cond, cp in copies:
        @pl.when(cond)
        def _():
            cp.wait()


def _group_mlp_kernel(sref, xg_hbm, w1_ref, b1_ref, w2_ref, b2_ref, og_hbm,
                      xgv_ref, ogv_ref, sem_in, sem_out):
    e = pl.program_id(0)
    s = pl.program_id(1)
    poff_e = pl.multiple_of(sref[0, e], BK)
    nb_e = sref[0, E + e]

    @pl.when(jnp.logical_and(e == 0, s == 0))
    def _():
        _start_guarded(_xg_rows_copy(sref, xg_hbm, xgv_ref, sem_in, 0))

    @pl.when(s == 0)
    def _():
        _wait_guarded(_xg_rows_copy(sref, xg_hbm, xgv_ref, sem_in, e))

        @pl.when(e < E - 1)
        def _():
            _start_guarded(_xg_rows_copy(sref, xg_hbm, xgv_ref, sem_in, e + 1))

    for cb in range(MAXCB):
        @pl.when(cb < nb_e)
        def _():
            rows = pl.ds(poff_e + cb * BK, BK)
            hs = jnp.dot(xgv_ref[rows, :], w1_ref[0],
                         preferred_element_type=jnp.float32) + b1_ref[0, 0]
            hs = _gelu_exact(hs)
            contrib = jnp.dot(hs, w2_ref[0],
                              preferred_element_type=jnp.float32)

            @pl.when(s == 0)
            def _():
                ogv_ref[rows, :] = contrib + b2_ref[0, 0]

            @pl.when(s > 0)
            def _():
                ogv_ref[rows, :] = ogv_ref[rows, :] + contrib

    @pl.when(s == NS - 1)
    def _():
        @pl.when(e > 0)
        def _():
            _wait_guarded(_og_rows_copy(sref, ogv_ref, og_hbm, sem_out, e - 1))

        _start_guarded(_og_rows_copy(sref, ogv_ref, og_hbm, sem_out, e))

        @pl.when(e == E - 1)
        def _():
            _wait_guarded(_og_rows_copy(sref, ogv_ref, og_hbm, sem_out, e))


def _group_mlp(scalars, xg, w1, b1r, w2, b2r):
    return pl.pallas_call(
        _group_mlp_kernel,
        grid_spec=pltpu.PrefetchScalarGridSpec(
            num_scalar_prefetch=1,
            grid=(E, NS),
            in_specs=[
                pl.BlockSpec(memory_space=pl.ANY),
                pl.BlockSpec((1, D, FS), lambda e, s, sc: (e, 0, s)),
                pl.BlockSpec((1, 1, FS), lambda e, s, sc: (e, 0, s)),
                pl.BlockSpec((1, FS, D), lambda e, s, sc: (e, s, 0)),
                pl.BlockSpec((1, 1, D), lambda e, s, sc: (e, 0, 0)),
            ],
            out_specs=pl.BlockSpec(memory_space=pl.ANY),
            scratch_shapes=[pltpu.VMEM((M, D), jnp.float32),
                            pltpu.VMEM((M, D), jnp.float32),
                            pltpu.SemaphoreType.DMA,
                            pltpu.SemaphoreType.DMA],
        ),
        out_shape=jax.ShapeDtypeStruct((M, D), jnp.float32),
    )(scalars, xg, w1, b1r, w2, b2r)


@functools.partial(
    pl.kernel, mesh=_SC_MESH,
    out_type=jax.ShapeDtypeStruct((N, D), jnp.float32),
    scratch_types=[
        pltpu.VMEM((CH, D), jnp.float32),
        pltpu.VMEM((CH, D), jnp.float32),
        pltpu.VMEM((CH,), jnp.int32),
        pltpu.VMEM((CH,), jnp.int32),
        pltpu.VMEM((CH, L), jnp.float32),
        pltpu.VMEM((CH, L), jnp.float32),
        pltpu.SemaphoreType.DMA,
    ])
def _gather_combine(og_hbm, d0_hbm, d1_hbm, g0_hbm, g1_hbm, out_hbm,
                    r0v, r1v, d0v, d1v, g0v, g1v, sem):
    wid = lax.axis_index("s") * 2 + lax.axis_index("c")
    base = wid * CH
    HH = CH // 4
    pltpu.sync_copy(d0_hbm.at[pl.ds(base, CH)], d0v)
    pltpu.sync_copy(d1_hbm.at[pl.ds(base, CH)], d1v)
    pltpu.sync_copy(g0_hbm.at[pl.ds(base, CH)], g0v)
    pltpu.sync_copy(g1_hbm.at[pl.ds(base, CH)], g1v)
    copies = []
    for hb in range(4):
        hsl = pl.ds(hb * HH, HH)
        copies.append((pltpu.async_copy(og_hbm.at[d0v.at[hsl]],
                                        r0v.at[hsl], sem),
                       pltpu.async_copy(og_hbm.at[d1v.at[hsl]],
                                        r1v.at[hsl], sem)))

    def row_body(r, _):
        ga = g0v[r, :]
        gb = g1v[r, :]
        for c in range(D // L):
            sl = pl.ds(c * L, L)
            r0v[r, sl] = ga * r0v[r, sl] + gb * r1v[r, sl]
        return 0

    outcp = []
    for hb in range(4):
        copies[hb][0].wait()
        copies[hb][1].wait()
        jax.lax.fori_loop(hb * HH, (hb + 1) * HH, row_body, 0)
        outcp.append(pltpu.async_copy(r0v.at[pl.ds(hb * HH, HH)],
                                      out_hbm.at[pl.ds(base + hb * HH, HH)],
                                      sem))
    for hb in range(4):
        outcp[hb].wait()


@jax.jit
def kernel(x, gate_w, gate_b, w1, b1, w2, b2):
    gb = gate_b.reshape(1, E)
    b1r = b1.reshape(E, 1, F)
    b2r = b2.reshape(E, 1, D)

    d0, d1, g0, g1, scalars = _router(x, gate_w, gb)
    d0f, d1f = d0.reshape(N), d1.reshape(N)

    xg = _dispatch(x, d0f, d1f)
    og = _group_mlp(scalars, xg, w1, b1r, w2, b2r)
    out = _gather_combine(og, d0f, d1f, g0, g1)
    return out
